# baseline (speedup 1.0000x reference)
import functools

import jax
import jax.numpy as jnp
from jax import lax
from jax.experimental import pallas as pl
from jax.experimental.pallas import tpu as pltpu
from jax.experimental.pallas import tpu_sc as plsc

N = 10000
E = 160000
F = 128
NT = 4

NC = 2
NS = 16
NW = NC * NS
CH = 64
E_PAD = 163840
NCH = E_PAD // (NW * CH)
IR = NCH * CH // 128
N_ACC = 10240

BN = 2000

BM = 1024
NB = 11
TB = NT * BM
ZB = (NB - 1) * TB
E2 = E // 128
EP2 = E_PAD // 128


def _mm_body(emb_ref, w_ref, b_ref, x_ref, ei_ref, et_ref,
             y_ref, gp_ref, dp_ref, mt_s, bt_s):
    i = pl.program_id(0)

    @pl.when(i == 0)
    def _():
        src = ei_ref[0]
        dstv = ei_ref[1]
        et = et_ref[...]
        g = ((src >> 10) << 12) | (et << 10) | (src & (BM - 1))
        npad = EP2 - E2 + 2
        r = lax.broadcasted_iota(jnp.int32, (npad, 128), 0)
        cc = lax.broadcasted_iota(jnp.int32, (npad, 128), 1)
        k = (E2 - 2 + r) * 128 + cc - E
        gpad = ZB + (k & (TB - 1))
        dpad = lax.rem(k + N, N)
        gp_ref[0:E2 - 2] = g[0:E2 - 2]
        dp_ref[0:E2 - 2] = dstv[0:E2 - 2]
        gp_ref[E2 - 2:EP2] = jnp.where(r < 2, jnp.concatenate(
            [g[E2 - 2:E2], gpad[2:]], axis=0), gpad)
        dp_ref[E2 - 2:EP2] = jnp.where(r < 2, jnp.concatenate(
            [dstv[E2 - 2:E2], dpad[2:]], axis=0), dpad)

    @pl.when(i == 0)
    def _():
        emb = emb_ref[...]
        m = jnp.max(emb, axis=-1, keepdims=True)
        ex = jnp.exp(emb - m)
        attr = ex / jnp.sum(ex, axis=-1, keepdims=True)
        wf = w_ref[...].reshape(NT, F * F)
        mt_s[...] = jnp.dot(attr, wf, preferred_element_type=jnp.float32
                            ).reshape(NT, F, F)
        bt_s[...] = jnp.dot(attr, b_ref[...], preferred_element_type=jnp.float32)

    @pl.when(i == NB - 1)
    def _():
        y_ref[...] = jnp.zeros_like(y_ref)

    @pl.when(i < NB - 1)
    def _():
        xb = x_ref[...]
        for t in range(NT):
            yt = lax.dot_general(xb, mt_s[t], (((1,), (1,)), ((), ())),
                                 preferred_element_type=jnp.float32)
            y_ref[t * BM:(t + 1) * BM, :] = yt + bt_s[t][None, :]


_mm = pl.pallas_call(
    _mm_body,
    grid=(NB,),
    in_specs=[
        pl.BlockSpec((NT, NT), lambda i: (0, 0)),
        pl.BlockSpec((NT, F, F), lambda i: (0, 0, 0)),
        pl.BlockSpec((NT, F), lambda i: (0, 0)),
        pl.BlockSpec((BM, F), lambda i: (jnp.minimum(i, NB - 2), 0)),
        pl.BlockSpec((2, E2, 128), lambda i: (0, 0, 0)),
        pl.BlockSpec((E2, 128), lambda i: (0, 0)),
    ],
    out_specs=(pl.BlockSpec((TB, F), lambda i: (i, 0)),
               pl.BlockSpec((EP2, 128), lambda i: (0, 0)),
               pl.BlockSpec((EP2, 128), lambda i: (0, 0))),
    out_shape=(jax.ShapeDtypeStruct((NB * TB, F), jnp.float32),
               jax.ShapeDtypeStruct((EP2, 128), jnp.int32),
               jax.ShapeDtypeStruct((EP2, 128), jnp.int32)),
    scratch_shapes=[
        pltpu.VMEM((NT, F, F), jnp.float32),
        pltpu.VMEM((NT, F), jnp.float32),
    ],
)


def _sc_body(y_hbm, g_hbm, d_hbm, z_hbm, out_hbm, gbuf, dbuf,
             r0, r1, r2, r3, acc,
             sg0, sg1, sg2, sg3, ss0, ss1, ss2, ss3):
    c = lax.axis_index("c")
    s = lax.axis_index("s")
    wid = s * NC + c
    rb = (r0, r1, r2, r3)
    sg = (sg0, sg1, sg2, sg3)
    ss = (ss0, ss1, ss2, ss3)
    nz = N_ACC // (NS * CH)
    with jax.named_scope("zero_phase"):
        for k in range(nz):
            pltpu.async_copy(z_hbm, acc.at[pl.ds((s * nz + k) * CH, CH)], ss0)
    pltpu.sync_copy(g_hbm.at[wid], gbuf)
    pltpu.sync_copy(d_hbm.at[wid], dbuf)

    NBUF = 4
    NI = NCH // NBUF

    def gsl(row, col):
        return gbuf.at[row, pl.ds(col * CH, CH)]

    def dsl(row, col):
        return dbuf.at[row, pl.ds(col * CH, CH)]

    pltpu.async_copy(y_hbm.at[gsl(0, 0)], rb[0], sg[0])
    pltpu.async_copy(y_hbm.at[gsl(0, 1)], rb[1], sg[1])
    with jax.named_scope("zero_wait"):
        for k in range(nz):
            pltpu.make_async_copy(z_hbm, acc.at[pl.ds((s * nz + k) * CH, CH)],
                                  ss0).wait()
        plsc.subcore_barrier()

    def body(i, carry):
        r0i = 2 * i
        for b in range(NBUF):
            row = r0i + b // 2
            col = b % 2
            bw = (b + 2) % NBUF
            rw = r0i + 1 + b // 2
            if b < 2:
                @pl.when(i > 0)
                def _():
                    pltpu.make_async_copy(rb[bw], acc.at[dsl(row, col)], ss[bw]).wait()
                pltpu.async_copy(y_hbm.at[gsl(rw, col)], rb[bw], sg[bw])
            else:
                @pl.when(i < NI - 1)
                def _():
                    pltpu.make_async_copy(rb[bw], acc.at[dsl(row, col)], ss[bw]).wait()
                    pltpu.async_copy(y_hbm.at[gsl(rw, col)], rb[bw], sg[bw])
            pltpu.make_async_copy(y_hbm.at[gsl(row, col)], rb[b], sg[b]).wait()
            pltpu.async_copy(rb[b], acc.at[dsl(row, col)], ss[b], add=True)
        return carry

    with jax.named_scope("gs_loop"):
        lax.fori_loop(0, NI, body, 0)
        for b in range(NBUF):
            pltpu.make_async_copy(rb[b], acc.at[dsl(IR - 2 + b // 2, b % 2)],
                                  ss[b]).wait()
    with jax.named_scope("post_barrier"):
        plsc.subcore_barrier()
    with jax.named_scope("copyout"):
        pltpu.sync_copy(acc.at[pl.ds(s * (N_ACC // NS), N_ACC // NS)],
                        out_hbm.at[c, pl.ds(s * (N_ACC // NS), N_ACC // NS)])


@functools.cache
def _get_sc():
    return functools.partial(
        pl.kernel,
        out_type=jax.ShapeDtypeStruct((NC, N_ACC, F), jnp.float32),
        mesh=plsc.VectorSubcoreMesh(core_axis_name="c", subcore_axis_name="s"),
        scratch_types=[
            pltpu.VMEM((IR, 128), jnp.int32),
            pltpu.VMEM((IR, 128), jnp.int32),
            pltpu.VMEM((CH, F), jnp.float32),
            pltpu.VMEM((CH, F), jnp.float32),
            pltpu.VMEM((CH, F), jnp.float32),
            pltpu.VMEM((CH, F), jnp.float32),
            pltpu.VMEM_SHARED((N_ACC, F), jnp.float32),
            pltpu.SemaphoreType.DMA,
            pltpu.SemaphoreType.DMA,
            pltpu.SemaphoreType.DMA,
            pltpu.SemaphoreType.DMA,
            pltpu.SemaphoreType.DMA,
            pltpu.SemaphoreType.DMA,
            pltpu.SemaphoreType.DMA,
            pltpu.SemaphoreType.DMA,
        ],
    )(_sc_body)


def _add_body(p_ref, o_ref):
    o_ref[...] = p_ref[0] + p_ref[1]


_add = pl.pallas_call(
    _add_body,
    grid=(N // BN,),
    in_specs=[pl.BlockSpec((NC, BN, F), lambda i: (0, i, 0))],
    out_specs=pl.BlockSpec((BN, F), lambda i: (i, 0)),
    out_shape=jax.ShapeDtypeStruct((N, F), jnp.float32),
)


def kernel(x, edge_index, edge_type, emb, W, b):
    y2, gp, dp = _mm(emb, W, b, x,
                     edge_index.reshape(2, E2, 128),
                     edge_type.reshape(E2, 128))
    g3 = gp.reshape(NW, IR, 128)
    d3 = dp.reshape(NW, IR, 128)
    z = jnp.zeros((CH, F), jnp.float32)
    partial = _get_sc()(y2, g3, d3, z)
    return _add(partial)


# speedup vs baseline: 1.1958x; 1.1958x over previous
import functools

import jax
import jax.numpy as jnp
from jax import lax
from jax.experimental import pallas as pl
from jax.experimental.pallas import tpu as pltpu
from jax.experimental.pallas import tpu_sc as plsc

N = 10000
E = 160000
F = 128
NT = 4

NC = 2
NS = 16
NW = NC * NS
CH = 64
E_PAD = 163840
NCH = E_PAD // (NW * CH)
IR = NCH * CH // 128
N_ACC = 10240

BN = 2000

BM = 1024
NB = 11
TB = NT * BM
ZB = (NB - 1) * TB
E2 = E // 128
EP2 = E_PAD // 128


def _mm_body(emb_ref, w_ref, b_ref, x_ref, ei_ref, et_ref,
             y_ref, gp_ref, dp_ref, mt_s, bt_s):
    i = pl.program_id(0)

    @pl.when(i == 0)
    def _():
        src = ei_ref[0]
        dstv = ei_ref[1]
        et = et_ref[...]
        g = ((src >> 10) << 12) | (et << 10) | (src & (BM - 1))
        npad = EP2 - E2 + 2
        r = lax.broadcasted_iota(jnp.int32, (npad, 128), 0)
        cc = lax.broadcasted_iota(jnp.int32, (npad, 128), 1)
        k = (E2 - 2 + r) * 128 + cc - E
        gpad = ZB + (k & (TB - 1))
        dpad = lax.rem(k + N, N)
        gp_ref[0:E2 - 2] = g[0:E2 - 2]
        dp_ref[0:E2 - 2] = dstv[0:E2 - 2]
        gp_ref[E2 - 2:EP2] = jnp.where(r < 2, jnp.concatenate(
            [g[E2 - 2:E2], gpad[2:]], axis=0), gpad)
        dp_ref[E2 - 2:EP2] = jnp.where(r < 2, jnp.concatenate(
            [dstv[E2 - 2:E2], dpad[2:]], axis=0), dpad)

    @pl.when(i == 0)
    def _():
        emb = emb_ref[...]
        m = jnp.max(emb, axis=-1, keepdims=True)
        ex = jnp.exp(emb - m)
        attr = ex / jnp.sum(ex, axis=-1, keepdims=True)
        wf = w_ref[...].reshape(NT, F * F)
        mt_s[...] = jnp.dot(attr, wf, preferred_element_type=jnp.float32
                            ).reshape(NT, F, F)
        bt_s[...] = jnp.dot(attr, b_ref[...], preferred_element_type=jnp.float32)

    @pl.when(i == NB - 1)
    def _():
        y_ref[...] = jnp.zeros_like(y_ref)

    @pl.when(i < NB - 1)
    def _():
        xb = x_ref[...]
        for t in range(NT):
            yt = lax.dot_general(xb, mt_s[t], (((1,), (1,)), ((), ())),
                                 preferred_element_type=jnp.float32)
            y_ref[t * BM:(t + 1) * BM, :] = yt + bt_s[t][None, :]


_mm = pl.pallas_call(
    _mm_body,
    grid=(NB,),
    in_specs=[
        pl.BlockSpec((NT, NT), lambda i: (0, 0)),
        pl.BlockSpec((NT, F, F), lambda i: (0, 0, 0)),
        pl.BlockSpec((NT, F), lambda i: (0, 0)),
        pl.BlockSpec((BM, F), lambda i: (jnp.minimum(i, NB - 2), 0)),
        pl.BlockSpec((2, E2, 128), lambda i: (0, 0, 0)),
        pl.BlockSpec((E2, 128), lambda i: (0, 0)),
    ],
    out_specs=(pl.BlockSpec((TB, F), lambda i: (i, 0)),
               pl.BlockSpec((EP2, 128), lambda i: (0, 0)),
               pl.BlockSpec((EP2, 128), lambda i: (0, 0))),
    out_shape=(jax.ShapeDtypeStruct((NB * TB, F), jnp.float32),
               jax.ShapeDtypeStruct((EP2, 128), jnp.int32),
               jax.ShapeDtypeStruct((EP2, 128), jnp.int32)),
    scratch_shapes=[
        pltpu.VMEM((NT, F, F), jnp.float32),
        pltpu.VMEM((NT, F), jnp.float32),
    ],
)


def _sc_body(y_hbm, g_hbm, d_hbm, z_hbm, out_hbm, gbuf, dbuf,
             r0, r1, r2, r3, acc,
             sg0, sg1, sg2, sg3, ss0, ss1, ss2, ss3):
    c = lax.axis_index("c")
    s = lax.axis_index("s")
    wid = s * NC + c
    rb = (r0, r1, r2, r3)
    sg = (sg0, sg1, sg2, sg3)
    ss = (ss0, ss1, ss2, ss3)
    with jax.named_scope("zero_phase"):
        pltpu.sync_copy(z_hbm, r0)
    nz = N_ACC // (NS * CH)
    for k in range(nz):
        pltpu.async_copy(r0, acc.at[pl.ds((s * nz + k) * CH, CH)], ss0)
    pltpu.sync_copy(g_hbm.at[wid], gbuf)
    pltpu.sync_copy(d_hbm.at[wid], dbuf)

    NBUF = 4
    NI = NCH // NBUF

    def gsl(row, col):
        return gbuf.at[row, pl.ds(col * CH, CH)]

    def dsl(row, col):
        return dbuf.at[row, pl.ds(col * CH, CH)]

    pltpu.async_copy(y_hbm.at[gsl(0, 0)], rb[0], sg[0])
    pltpu.async_copy(y_hbm.at[gsl(0, 1)], rb[1], sg[1])
    with jax.named_scope("zero_wait"):
        for k in range(nz):
            pltpu.make_async_copy(r0, acc.at[pl.ds((s * nz + k) * CH, CH)], ss0).wait()
        plsc.subcore_barrier()

    def body(i, carry):
        r0i = 2 * i
        for b in range(NBUF):
            row = r0i + b // 2
            col = b % 2
            bw = (b + 2) % NBUF
            rw = r0i + 1 + b // 2
            if b < 2:
                @pl.when(i > 0)
                def _():
                    pltpu.make_async_copy(rb[bw], acc.at[dsl(row, col)], ss[bw]).wait()
                pltpu.async_copy(y_hbm.at[gsl(rw, col)], rb[bw], sg[bw])
            else:
                @pl.when(i < NI - 1)
                def _():
                    pltpu.make_async_copy(rb[bw], acc.at[dsl(row, col)], ss[bw]).wait()
                    pltpu.async_copy(y_hbm.at[gsl(rw, col)], rb[bw], sg[bw])
            pltpu.make_async_copy(y_hbm.at[gsl(row, col)], rb[b], sg[b]).wait()
            pltpu.async_copy(rb[b], acc.at[dsl(row, col)], ss[b], add=True)
        return carry

    with jax.named_scope("gs_loop"):
        lax.fori_loop(0, NI, body, 0)
        for b in range(NBUF):
            pltpu.make_async_copy(rb[b], acc.at[dsl(IR - 2 + b // 2, b % 2)],
                                  ss[b]).wait()
    with jax.named_scope("post_barrier"):
        plsc.subcore_barrier()
    with jax.named_scope("copyout"):
        pltpu.sync_copy(acc.at[pl.ds(s * (N_ACC // NS), N_ACC // NS)],
                        out_hbm.at[c, pl.ds(s * (N_ACC // NS), N_ACC // NS)])


@functools.cache
def _get_sc():
    return functools.partial(
        pl.kernel,
        out_type=jax.ShapeDtypeStruct((NC, N_ACC, F), jnp.float32),
        mesh=plsc.VectorSubcoreMesh(core_axis_name="c", subcore_axis_name="s"),
        scratch_types=[
            pltpu.VMEM((IR, 128), jnp.int32),
            pltpu.VMEM((IR, 128), jnp.int32),
            pltpu.VMEM((CH, F), jnp.float32),
            pltpu.VMEM((CH, F), jnp.float32),
            pltpu.VMEM((CH, F), jnp.float32),
            pltpu.VMEM((CH, F), jnp.float32),
            pltpu.VMEM_SHARED((N_ACC, F), jnp.float32),
            pltpu.SemaphoreType.DMA,
            pltpu.SemaphoreType.DMA,
            pltpu.SemaphoreType.DMA,
            pltpu.SemaphoreType.DMA,
            pltpu.SemaphoreType.DMA,
            pltpu.SemaphoreType.DMA,
            pltpu.SemaphoreType.DMA,
            pltpu.SemaphoreType.DMA,
        ],
    )(_sc_body)


def _add_body(p_ref, o_ref):
    o_ref[...] = p_ref[0] + p_ref[1]


_add = pl.pallas_call(
    _add_body,
    grid=(N // BN,),
    in_specs=[pl.BlockSpec((NC, BN, F), lambda i: (0, i, 0))],
    out_specs=pl.BlockSpec((BN, F), lambda i: (i, 0)),
    out_shape=jax.ShapeDtypeStruct((N, F), jnp.float32),
)


def kernel(x, edge_index, edge_type, emb, W, b):
    y2, gp, dp = _mm(emb, W, b, x,
                     edge_index.reshape(2, E2, 128),
                     edge_type.reshape(E2, 128))
    g3 = gp.reshape(NW, IR, 128)
    d3 = dp.reshape(NW, IR, 128)
    z = jnp.zeros((CH, F), jnp.float32)
    partial = _get_sc()(y2, g3, d3, z)
    return _add(partial)
